# Initial kernel scaffold; baseline (speedup 1.0000x reference)
import jax
import jax.numpy as jnp
from jax.experimental import pallas as pl


def kernel(pred, gt):
    raise NotImplementedError("write your pallas kernel here")


# speedup vs baseline: 1.9681x; 1.9681x over previous
import functools

import jax
import jax.numpy as jnp
from jax.experimental import pallas as pl
from jax.experimental.pallas import tpu as pltpu

RADIUS2 = 25.0
NUM_CLASSES = 10
BIG = 1e30


def _body(pred_ref, gt_ref, out_ref, best_ref, bidx_ref, tp_ref, *, nb_total, pb, n_b, n_p, n_t):
    b = pl.program_id(0)
    nb = pl.program_id(1)

    @pl.when(jnp.logical_and(b == 0, nb == 0))
    def _init_tp():
        tp_ref[0, 0] = 0.0

    @pl.when(nb == 0)
    def _init_sample():
        best_ref[...] = jnp.full((1, 128), BIG, jnp.float32)
        bidx_ref[...] = jnp.zeros((1, 128), jnp.int32)

    pc = pred_ref[0, :, 0:1]
    px = pred_ref[0, :, 1:2]
    py = pred_ref[0, :, 2:3]
    tc = gt_ref[0, 0:1, :]
    tx = gt_ref[0, 1:2, :]
    ty = gt_ref[0, 2:3, :]

    dx = px - tx
    dy = py - ty
    d2 = dx * dx + dy * dy
    valid = jnp.logical_and(pc == tc, d2 <= RADIUS2)
    key = jnp.where(valid, d2, BIG)

    blk_min = jnp.min(key, axis=0, keepdims=True)
    rows = jax.lax.broadcasted_iota(jnp.int32, (pb, 128), 0) + nb * pb
    blk_idx = jnp.min(jnp.where(key == blk_min, rows, jnp.int32(2**30)),
                      axis=0, keepdims=True)

    upd = blk_min < best_ref[...]
    best_ref[...] = jnp.where(upd, blk_min, best_ref[...])
    bidx_ref[...] = jnp.where(upd, blk_idx, bidx_ref[...])

    @pl.when(nb == nb_total - 1)
    def _finalize():
        best = best_ref[...]
        matched = best <= RADIUS2
        lane = jax.lax.broadcasted_iota(jnp.int32, (1, 128), 1)
        fkey = jnp.where(matched, bidx_ref[...], -1 - lane).astype(jnp.float32)
        r0 = jax.lax.broadcasted_iota(jnp.int32, (128, 128), 0)
        r1 = jax.lax.broadcasted_iota(jnp.int32, (128, 128), 1)
        ident = (r0 == r1).astype(jnp.float32)
        fkey_col = jax.lax.dot_general(
            ident, fkey, (((1,), (1,)), ((), ())),
            preferred_element_type=jnp.float32)
        eq = fkey_col == fkey
        earlier = r1 < r0
        dup = jnp.max(jnp.logical_and(eq, earlier).astype(jnp.float32),
                      axis=1, keepdims=True)
        n_matched = jnp.sum(matched.astype(jnp.float32))
        n_dup = jnp.sum(dup)
        tp_ref[0, 0] = tp_ref[0, 0] + (n_matched - n_dup)

        @pl.when(b == n_b - 1)
        def _loss():
            tp = tp_ref[0, 0]
            fp = jnp.float32(n_b * n_p) - tp
            fn = jnp.float32(n_b * n_t) - tp
            prec = (tp + 1e-06) / (tp + 1e-06 + fp + 1e-06)
            rec = (tp + 1e-06) / (tp + fn + 1e-06)
            f1 = 2.0 * prec * rec / (prec + rec)
            out_ref[...] = jnp.full((1, 1), 1.0 - f1, jnp.float32)


def kernel(pred, gt):
    B, P, _ = pred.shape
    T = gt.shape[1]
    PB = 2000
    NB = P // PB

    gt_pad = jnp.pad(gt, ((0, 0), (0, 128 - T), (0, 0)),
                     constant_values=-1.0)
    gt_t = jnp.transpose(gt_pad, (0, 2, 1))

    body = functools.partial(_body, nb_total=NB, pb=PB, n_b=B, n_p=P, n_t=T)
    out = pl.pallas_call(
        body,
        grid=(B, NB),
        in_specs=[
            pl.BlockSpec((1, PB, 3), lambda b, nb: (b, nb, 0)),
            pl.BlockSpec((1, 3, 128), lambda b, nb: (b, 0, 0)),
        ],
        out_specs=pl.BlockSpec((1, 1), lambda b, nb: (0, 0)),
        out_shape=jax.ShapeDtypeStruct((1, 1), jnp.float32),
        scratch_shapes=[
            pltpu.VMEM((1, 128), jnp.float32),
            pltpu.VMEM((1, 128), jnp.int32),
            pltpu.SMEM((1, 1), jnp.float32),
        ],
    )(pred, gt_t)
    return jnp.reshape(out, ())
